# baseline (speedup 1.0000x reference)
import functools

import jax
import jax.numpy as jnp
from jax import lax
from jax.experimental import pallas as pl
from jax.experimental.pallas import tpu as pltpu
from jax.experimental.pallas import tpu_sc as plsc
from jax.sharding import PartitionSpec

_B, _N, _K, _D = 4, 4096, 1024, 64
_NDEV = 2
_BL = _B // _NDEV
_PL = _BL * _N
_ROW_BLK = 2048
_STEPS = _PL // _ROW_BLK

_NC, _NS = 2, 16
_NW = _NC * _NS
_DBLK = 8
_NH = _NW // (_BL * (_D // _DBLK))
_NBLK = _N // _NH
_L = 16


def _argmin_body(pts_ref, keys_ref, ids_ref):
    px = pts_ref[0:1, :]
    py = pts_ref[1:2, :]
    pz = pts_ref[2:3, :]
    kx = keys_ref[:, 0:1]
    ky = keys_ref[:, 1:2]
    kz = keys_ref[:, 2:3]
    d0 = px - kx
    acc = d0 * d0
    d1 = py - ky
    acc = acc + d1 * d1
    d2 = pz - kz
    acc = acc + d2 * d2
    m = jnp.min(acc, axis=0, keepdims=True)
    io = lax.broadcasted_iota(jnp.int32, (_K, _ROW_BLK), 0)
    idx = jnp.min(jnp.where(acc <= m, io, _K), axis=0, keepdims=True)
    ids_ref[...] = idx.reshape(1, 1, _ROW_BLK)


def _tc_argmin(pts_t, keys):
    return pl.pallas_call(
        _argmin_body,
        grid=(_STEPS,),
        in_specs=[
            pl.BlockSpec((3, _ROW_BLK), lambda i: (0, i)),
            pl.BlockSpec((_K, 3), lambda i: (0, 0)),
        ],
        out_specs=pl.BlockSpec((1, 1, _ROW_BLK), lambda i: (i, 0, 0)),
        out_shape=jax.ShapeDtypeStruct((_STEPS, 1, _ROW_BLK), jnp.int32),
    )(pts_t, keys)


@functools.partial(
    pl.kernel,
    out_type=jax.ShapeDtypeStruct((_BL, _D, _N), jnp.float32),
    mesh=plsc.VectorSubcoreMesh(core_axis_name="c", subcore_axis_name="s"),
    scratch_types=[
        pltpu.VMEM((_DBLK, _K), jnp.float32),
        pltpu.VMEM((_NBLK,), jnp.int32),
        pltpu.VMEM((_DBLK, _NBLK), jnp.float32),
    ],
    compiler_params=pltpu.CompilerParams(
        use_tc_tiling_on_sc=False, needs_layout_passes=False
    ),
)
def _sc_gather_t(values_t_hbm, idx_hbm, out_hbm, vt_v, ids_v, out_v):
    wid = lax.axis_index("s") * _NC + lax.axis_index("c")
    b = wid // (_NW // _BL)
    t = (wid // _NH) % (_D // _DBLK)
    nh = wid % _NH
    pltpu.sync_copy(values_t_hbm.at[pl.ds(t * _DBLK, _DBLK)], vt_v)
    pltpu.sync_copy(
        idx_hbm.at[pl.ds(b * _N + nh * _NBLK, _NBLK)], ids_v
    )

    def body(i, carry):
        n0 = i * (4 * _L)
        for g in range(4):
            id16 = ids_v[pl.ds(n0 + g * _L, _L)]
            for d in range(_DBLK):
                vals = plsc.load_gather(
                    vt_v, [jnp.full((_L,), d, jnp.int32), id16]
                )
                out_v[d, pl.ds(n0 + g * _L, _L)] = vals
        return carry

    lax.fori_loop(0, _NBLK // (4 * _L), body, None)
    pltpu.sync_copy(
        out_v, out_hbm.at[b, pl.ds(t * _DBLK, _DBLK), pl.ds(nh * _NBLK, _NBLK)]
    )


def _device_local(pc, keys, values):
    pts_t = jnp.transpose(pc, (2, 0, 1)).reshape(3, _PL)
    ids = _tc_argmin(pts_t, keys)
    feats_t = _sc_gather_t(values.T, ids.reshape(_PL))
    return jnp.transpose(feats_t, (0, 2, 1))


def kernel(pointcloud, keys, values):
    mesh = jax.make_mesh((_NDEV,), ("x",))
    pc = jax.reshard(pointcloud, jax.NamedSharding(mesh, PartitionSpec("x")))
    keys_r = jax.reshard(keys, jax.NamedSharding(mesh, PartitionSpec()))
    values_r = jax.reshard(values, jax.NamedSharding(mesh, PartitionSpec()))
    feats = jax.shard_map(
        _device_local,
        mesh=mesh,
        in_specs=(PartitionSpec("x"), PartitionSpec(), PartitionSpec()),
        out_specs=PartitionSpec("x"),
        check_vma=False,
    )(pc, keys_r, values_r)
    return feats, pointcloud


# speedup vs baseline: 6.2502x; 6.2502x over previous
import functools

import jax
import jax.numpy as jnp
from jax import lax
from jax.experimental import pallas as pl
from jax.experimental.pallas import tpu as pltpu
from jax.experimental.pallas import tpu_sc as plsc

_B, _N, _K, _D = 4, 4096, 1024, 64
_P = _B * _N
_ROW_BLK = 2048
_STEPS = _P // _ROW_BLK

_NC, _NS = 2, 16
_NW = _NC * _NS
_DBLK = 8
_L = 16


def _argmin_body(pts_ref, keys_ref, ids_ref):
    px = pts_ref[0:1, :]
    py = pts_ref[1:2, :]
    pz = pts_ref[2:3, :]
    kx = keys_ref[:, 0:1]
    ky = keys_ref[:, 1:2]
    kz = keys_ref[:, 2:3]
    d0 = px - kx
    acc = d0 * d0
    d1 = py - ky
    acc = acc + d1 * d1
    d2 = pz - kz
    acc = acc + d2 * d2
    m = jnp.min(acc, axis=0, keepdims=True)
    io = lax.broadcasted_iota(jnp.int32, (_K, _ROW_BLK), 0)
    idx = jnp.min(jnp.where(acc <= m, io, _K), axis=0, keepdims=True)
    ids_ref[...] = idx.reshape(1, 1, _ROW_BLK)


def _tc_argmin(pts_t, keys):
    return pl.pallas_call(
        _argmin_body,
        grid=(_STEPS,),
        in_specs=[
            pl.BlockSpec((3, _ROW_BLK), lambda i: (0, i)),
            pl.BlockSpec((_K, 3), lambda i: (0, 0)),
        ],
        out_specs=pl.BlockSpec((1, 1, _ROW_BLK), lambda i: (i, 0, 0)),
        out_shape=jax.ShapeDtypeStruct((_STEPS, 1, _ROW_BLK), jnp.int32),
    )(pts_t, keys)


@functools.partial(
    pl.kernel,
    out_type=jax.ShapeDtypeStruct((_B, _D, _N), jnp.float32),
    mesh=plsc.VectorSubcoreMesh(core_axis_name="c", subcore_axis_name="s"),
    scratch_types=[
        pltpu.VMEM((_DBLK, _K), jnp.float32),
        pltpu.VMEM((_N,), jnp.int32),
        pltpu.VMEM((_DBLK, _N), jnp.float32),
    ],
    compiler_params=pltpu.CompilerParams(
        use_tc_tiling_on_sc=False, needs_layout_passes=False
    ),
)
def _sc_gather_t(values_t_hbm, idx_hbm, out_hbm, vt_v, ids_v, out_v):
    wid = lax.axis_index("s") * _NC + lax.axis_index("c")
    b = wid // (_D // _DBLK)
    t = wid % (_D // _DBLK)
    pltpu.sync_copy(values_t_hbm.at[pl.ds(t * _DBLK, _DBLK)], vt_v)
    pltpu.sync_copy(idx_hbm.at[pl.ds(b * _N, _N)], ids_v)

    def body(i, carry):
        n0 = i * (4 * _L)
        for g in range(4):
            id16 = ids_v[pl.ds(n0 + g * _L, _L)]
            for d in range(_DBLK):
                vals = plsc.load_gather(
                    vt_v, [jnp.full((_L,), d, jnp.int32), id16]
                )
                out_v[d, pl.ds(n0 + g * _L, _L)] = vals
        return carry

    lax.fori_loop(0, _N // (4 * _L), body, None)
    pltpu.sync_copy(out_v, out_hbm.at[b, pl.ds(t * _DBLK, _DBLK)])


def kernel(pointcloud, keys, values):
    pts_t = jnp.transpose(pointcloud, (2, 0, 1)).reshape(3, _P)
    ids = _tc_argmin(pts_t, keys)
    feats_t = _sc_gather_t(values.T, ids.reshape(_P))
    return jnp.transpose(feats_t, (0, 2, 1)), pointcloud


# speedup vs baseline: 6.8562x; 1.0970x over previous
import functools

import jax
import jax.numpy as jnp
from jax import lax
from jax.experimental import pallas as pl
from jax.experimental.pallas import tpu as pltpu
from jax.experimental.pallas import tpu_sc as plsc

_B, _N, _K, _D = 4, 4096, 1024, 64
_P = _B * _N
_ROW_BLK = 2048
_STEPS = _P // _ROW_BLK

_NC, _NS = 2, 16
_NW = _NC * _NS
_DBLK = 8
_L = 16


def _argmin_body(pts_ref, keys_ref, ids_ref):
    px = pts_ref[0:1, :]
    py = pts_ref[1:2, :]
    pz = pts_ref[2:3, :]
    kx = keys_ref[:, 0:1]
    ky = keys_ref[:, 1:2]
    kz = keys_ref[:, 2:3]
    d0 = px - kx
    acc = d0 * d0
    d1 = py - ky
    acc = acc + d1 * d1
    d2 = pz - kz
    acc = acc + d2 * d2
    m = jnp.min(acc, axis=0, keepdims=True)
    io = lax.broadcasted_iota(jnp.int32, (_K, _ROW_BLK), 0)
    idx = jnp.min(jnp.where(acc <= m, io, _K), axis=0, keepdims=True)
    ids_ref[...] = idx.reshape(1, 1, _ROW_BLK)


def _tc_argmin(pts_t, keys):
    return pl.pallas_call(
        _argmin_body,
        grid=(_STEPS,),
        in_specs=[
            pl.BlockSpec((3, _ROW_BLK), lambda i: (0, i)),
            pl.BlockSpec((_K, 3), lambda i: (0, 0)),
        ],
        out_specs=pl.BlockSpec((1, 1, _ROW_BLK), lambda i: (i, 0, 0)),
        out_shape=jax.ShapeDtypeStruct((_STEPS, 1, _ROW_BLK), jnp.int32),
    )(pts_t, keys)


@functools.partial(
    pl.kernel,
    out_type=jax.ShapeDtypeStruct((_B, _D, _N), jnp.float32),
    mesh=plsc.VectorSubcoreMesh(core_axis_name="c", subcore_axis_name="s"),
    scratch_types=[
        pltpu.VMEM((_DBLK, _K), jnp.float32),
        pltpu.VMEM((_N,), jnp.int32),
        pltpu.VMEM((_DBLK, _N), jnp.float32),
    ],
    compiler_params=pltpu.CompilerParams(
        use_tc_tiling_on_sc=False, needs_layout_passes=False
    ),
)
def _sc_gather_t(values_t_hbm, idx_hbm, out_hbm, vt_v, ids_v, out_v):
    wid = lax.axis_index("s") * _NC + lax.axis_index("c")
    b = wid // (_D // _DBLK)
    t = wid % (_D // _DBLK)
    pltpu.sync_copy(values_t_hbm.at[pl.ds(t * _DBLK, _DBLK)], vt_v)
    pltpu.sync_copy(idx_hbm.at[pl.ds(b * _N, _N)], ids_v)

    @plsc.parallel_loop(0, _N // _L, unroll=4)
    def body(i):
        n0 = i * _L
        id16 = ids_v[pl.ds(n0, _L)]
        for d in range(_DBLK):
            vals = plsc.load_gather(
                vt_v, [jnp.full((_L,), d, jnp.int32), id16]
            )
            out_v[d, pl.ds(n0, _L)] = vals
    pltpu.sync_copy(out_v, out_hbm.at[b, pl.ds(t * _DBLK, _DBLK)])


def kernel(pointcloud, keys, values):
    pts_t = jnp.transpose(pointcloud, (2, 0, 1)).reshape(3, _P)
    ids = _tc_argmin(pts_t, keys)
    feats_t = _sc_gather_t(values.T, ids.reshape(_P))
    return jnp.transpose(feats_t, (0, 2, 1)), pointcloud


# speedup vs baseline: 7.5844x; 1.1062x over previous
import functools

import jax
import jax.numpy as jnp
from jax import lax
from jax.experimental import pallas as pl
from jax.experimental.pallas import tpu as pltpu
from jax.experimental.pallas import tpu_sc as plsc

_B, _N, _K, _D = 4, 4096, 1024, 64
_P = _B * _N
_ROW_BLK = 2048
_STEPS = _P // _ROW_BLK

_NC, _NS = 2, 16
_NW = _NC * _NS
_DBLK = 8
_L = 16


def _argmin_body(pts_ref, keys_ref, ids_ref):
    px = pts_ref[0:1, :]
    py = pts_ref[1:2, :]
    pz = pts_ref[2:3, :]
    kx = keys_ref[:, 0:1]
    ky = keys_ref[:, 1:2]
    kz = keys_ref[:, 2:3]
    d0 = px - kx
    acc = d0 * d0
    d1 = py - ky
    acc = acc + d1 * d1
    d2 = pz - kz
    acc = acc + d2 * d2
    m = jnp.min(acc, axis=0, keepdims=True)
    io = lax.broadcasted_iota(jnp.int32, (_K, _ROW_BLK), 0)
    idx = jnp.min(jnp.where(acc <= m, io, _K), axis=0, keepdims=True)
    ids_ref[...] = idx.reshape(1, 1, _ROW_BLK)


def _tc_argmin(pts_t, keys):
    return pl.pallas_call(
        _argmin_body,
        grid=(_STEPS,),
        in_specs=[
            pl.BlockSpec((3, _ROW_BLK), lambda i: (0, i)),
            pl.BlockSpec((_K, 3), lambda i: (0, 0)),
        ],
        out_specs=pl.BlockSpec((1, 1, _ROW_BLK), lambda i: (i, 0, 0)),
        out_shape=jax.ShapeDtypeStruct((_STEPS, 1, _ROW_BLK), jnp.int32),
    )(pts_t, keys)


@functools.partial(
    pl.kernel,
    out_type=jax.ShapeDtypeStruct((_B, _D, _N), jnp.float32),
    mesh=plsc.VectorSubcoreMesh(core_axis_name="c", subcore_axis_name="s"),
    scratch_types=[
        pltpu.VMEM((_DBLK, _K), jnp.float32),
        pltpu.VMEM((_N,), jnp.int32),
        pltpu.VMEM((_DBLK, _N), jnp.float32),
    ],
    compiler_params=pltpu.CompilerParams(
        use_tc_tiling_on_sc=True, needs_layout_passes=False
    ),
)
def _sc_gather_t(values_t_hbm, idx_hbm, out_hbm, vt_v, ids_v, out_v):
    wid = lax.axis_index("s") * _NC + lax.axis_index("c")
    b = wid // (_D // _DBLK)
    t = wid % (_D // _DBLK)
    pltpu.sync_copy(values_t_hbm.at[pl.ds(t * _DBLK, _DBLK)], vt_v)
    pltpu.sync_copy(idx_hbm.at[pl.ds(b * _N, _N)], ids_v)

    @plsc.parallel_loop(0, _N // _L, unroll=4)
    def body(i):
        n0 = i * _L
        id16 = ids_v[pl.ds(n0, _L)]
        for d in range(_DBLK):
            vals = plsc.load_gather(
                vt_v, [jnp.full((_L,), d, jnp.int32), id16]
            )
            out_v[d, pl.ds(n0, _L)] = vals
    pltpu.sync_copy(out_v, out_hbm.at[b, pl.ds(t * _DBLK, _DBLK)])


def kernel(pointcloud, keys, values):
    pts_t = jnp.transpose(pointcloud, (2, 0, 1)).reshape(3, _P)
    ids = _tc_argmin(pts_t, keys)
    feats_t = _sc_gather_t(values.T, ids.reshape(_P))
    return jnp.transpose(feats_t, (0, 2, 1)), pointcloud
